# Initial kernel scaffold; baseline (speedup 1.0000x reference)
import jax
import jax.numpy as jnp
from jax.experimental import pallas as pl


def kernel(x, edge_index, W_first, b_first, W_rel1, b_rel1, W_root1, W_rel2, b_rel2, W_root2, fuse_weight, W_out, b_out):
    raise NotImplementedError("write your pallas kernel here")


# speedup vs baseline: 9.3383x; 9.3383x over previous
import functools

import jax
import jax.numpy as jnp
from jax import lax
from jax.experimental import pallas as pl
from jax.experimental.pallas import tpu as pltpu
from jax.experimental.pallas import tpu_sc as plsc

N = 100000
E = 1600000
F_IN = 128
H = 32
HH = 16
C = 2

NC = 2
NS = 16

ROWS_PER_SUB_N = 6272
NP = NS * ROWS_PER_SUB_N
EROW = 128
EP_ROWS = 12544
ROWS_PER_SUB_E = EP_ROWS // NS
SUPER = 8
NSUPER = ROWS_PER_SUB_E // SUPER

RB = 2000
NBLK = N // RB


def _segsum_body(hp_flat, srcb, dstb, out, idx_s, idx_d, rows, acc, gsem, ssem):
  c = lax.axis_index("c")
  s = lax.axis_index("s")

  @pl.loop(0, EROW)
  def _(i):
    rows[0, i, :] = jnp.zeros((16,), jnp.float32)

  @pl.loop(0, ROWS_PER_SUB_N // EROW)
  def _(k):
    pltpu.sync_copy(rows.at[0], acc.at[pl.ds(s * ROWS_PER_SUB_N + k * EROW, EROW)])

  plsc.subcore_barrier()

  base_row = s * ROWS_PER_SUB_E

  @pl.loop(0, NSUPER)
  def _(k):
    r0 = base_row + k * SUPER
    pltpu.sync_copy(srcb.at[c].at[pl.ds(r0, SUPER)], idx_s)
    pltpu.sync_copy(dstb.at[pl.ds(r0, SUPER)], idx_d)
    gathers = [
        pltpu.async_copy(hp_flat.at[idx_s.at[j]], rows.at[j], gsem)
        for j in range(SUPER)
    ]
    for cp in gathers:
      cp.wait()
    scatters = [
        pltpu.async_copy(rows.at[j], acc.at[idx_d.at[j]], ssem, add=True)
        for j in range(SUPER)
    ]
    for cp in scatters:
      cp.wait()

  plsc.subcore_barrier()
  pltpu.sync_copy(
      acc.at[pl.ds(s * ROWS_PER_SUB_N, ROWS_PER_SUB_N)],
      out.at[c].at[pl.ds(s * ROWS_PER_SUB_N, ROWS_PER_SUB_N)],
  )


@functools.cache
def _build_segsum():
  return pl.kernel(
      _segsum_body,
      out_type=jax.ShapeDtypeStruct((NC, NP, HH), jnp.float32),
      mesh=plsc.VectorSubcoreMesh(
          core_axis_name="c", subcore_axis_name="s",
          num_cores=NC, num_subcores=NS
      ),
      scratch_types=[
          pltpu.VMEM((SUPER, EROW), jnp.int32),
          pltpu.VMEM((SUPER, EROW), jnp.int32),
          pltpu.VMEM((SUPER, EROW, HH), jnp.float32),
          pltpu.VMEM_SHARED((NP, HH), jnp.float32),
          pltpu.SemaphoreType.DMA,
          pltpu.SemaphoreType.DMA,
      ],
      compiler_params=pltpu.CompilerParams(use_tc_tiling_on_sc=False),
  )


def _segsum(hp_flat, src_both, dst_r):
  return _build_segsum()(hp_flat, src_both, dst_r)


def _first_body(x_ref, w_ref, b_ref, o_ref):
  y = jnp.dot(x_ref[...], w_ref[...], preferred_element_type=jnp.float32)
  y = jnp.maximum(y + b_ref[...], 0.0)
  o_ref[0] = y[:, :HH]
  o_ref[1] = y[:, HH:]


_first = pl.pallas_call(
    _first_body,
    out_shape=jax.ShapeDtypeStruct((NC, NP, HH), jnp.float32),
    grid=(NBLK,),
    in_specs=[
        pl.BlockSpec((RB, F_IN), lambda i: (i, 0)),
        pl.BlockSpec((F_IN, H), lambda i: (0, 0)),
        pl.BlockSpec((1, H), lambda i: (0, 0)),
    ],
    out_specs=pl.BlockSpec((NC, RB, HH), lambda i: (0, i, 0)),
)


def _conv_body(agg_ref, h_ref, hf_ref, wrel_ref, brel_ref, wroot_ref, fw_ref,
               o_ref, *, fuse_idx):
  agg = jnp.concatenate([agg_ref[0], agg_ref[1]], axis=1)
  h = jnp.concatenate([h_ref[0], h_ref[1]], axis=1)
  hf = jnp.concatenate([hf_ref[0], hf_ref[1]], axis=1)
  y = jnp.dot(agg, wrel_ref[...], preferred_element_type=jnp.float32)
  y = y + brel_ref[...]
  y = y + jnp.dot(h, wroot_ref[...], preferred_element_type=jnp.float32)
  y = jnp.maximum(y, 0.0) + fw_ref[0, fuse_idx] * hf
  o_ref[0] = y[:, :HH]
  o_ref[1] = y[:, HH:]


def _make_conv(fuse_idx):
  return pl.pallas_call(
      functools.partial(_conv_body, fuse_idx=fuse_idx),
      out_shape=jax.ShapeDtypeStruct((NC, NP, HH), jnp.float32),
      grid=(NBLK,),
      in_specs=[
          pl.BlockSpec((NC, RB, HH), lambda i: (0, i, 0)),
          pl.BlockSpec((NC, RB, HH), lambda i: (0, i, 0)),
          pl.BlockSpec((NC, RB, HH), lambda i: (0, i, 0)),
          pl.BlockSpec((H, H), lambda i: (0, 0)),
          pl.BlockSpec((1, H), lambda i: (0, 0)),
          pl.BlockSpec((H, H), lambda i: (0, 0)),
          pl.BlockSpec((1, 2), lambda i: (0, 0)),
      ],
      out_specs=pl.BlockSpec((NC, RB, HH), lambda i: (0, i, 0)),
  )


_conv1 = _make_conv(0)
_conv2 = _make_conv(1)


def _out_body(h2_ref, wout_ref, bout_ref, o_ref):
  h2 = jnp.concatenate([h2_ref[0], h2_ref[1]], axis=1)
  logits = jnp.dot(h2, wout_ref[...], preferred_element_type=jnp.float32)
  logits = logits + bout_ref[...]
  m = jnp.max(logits, axis=1, keepdims=True)
  lse = m + jnp.log(jnp.sum(jnp.exp(logits - m), axis=1, keepdims=True))
  o_ref[...] = logits - lse


_out = pl.pallas_call(
    _out_body,
    out_shape=jax.ShapeDtypeStruct((N, C), jnp.float32),
    grid=(NBLK,),
    in_specs=[
        pl.BlockSpec((NC, RB, HH), lambda i: (0, i, 0)),
        pl.BlockSpec((H, C), lambda i: (0, 0)),
        pl.BlockSpec((1, C), lambda i: (0, 0)),
    ],
    out_specs=pl.BlockSpec((RB, C), lambda i: (i, 0)),
)


def kernel(x, edge_index, W_first, b_first, W_rel1, b_rel1, W_root1,
           W_rel2, b_rel2, W_root2, fuse_weight, W_out, b_out):
  EP = EP_ROWS * EROW
  src = edge_index[0]
  dst = edge_index[1]
  src_p = jnp.concatenate([src, jnp.zeros((EP - E,), jnp.int32)])
  dst_p = jnp.concatenate([dst, jnp.full((EP - E,), N, jnp.int32)])
  src_r = src_p.reshape(EP_ROWS, EROW)
  src_both = jnp.stack([src_r, src_r + NP])
  dst_r = dst_p.reshape(EP_ROWS, EROW)

  b_first2 = b_first.reshape(1, H)
  b_rel1_2 = b_rel1.reshape(1, H)
  b_rel2_2 = b_rel2.reshape(1, H)
  fw2 = fuse_weight.reshape(1, 2)
  b_out2 = b_out.reshape(1, C)

  hp = _first(x, W_first.T, b_first2)
  agg1 = _segsum(hp.reshape(NC * NP, HH), src_both, dst_r)
  h1p = _conv1(agg1, hp, hp, W_rel1.T, b_rel1_2, W_root1.T, fw2)
  agg2 = _segsum(h1p.reshape(NC * NP, HH), src_both, dst_r)
  h2p = _conv2(agg2, h1p, hp, W_rel2.T, b_rel2_2, W_root2.T, fw2)
  return _out(h2p, W_out.T, b_out2)


# speedup vs baseline: 11.6524x; 1.2478x over previous
import functools

import jax
import jax.numpy as jnp
from jax import lax
from jax.experimental import pallas as pl
from jax.experimental.pallas import tpu as pltpu
from jax.experimental.pallas import tpu_sc as plsc

N = 100000
E = 1600000
F_IN = 128
H = 32
HH = 16
C = 2

NC = 2
NS = 16

ROWS_PER_SUB_N = 6272
NP = NS * ROWS_PER_SUB_N
NPK = NP * HH // 128
EROW = 128
EP_ROWS = 12544
ROWS_PER_SUB_E = EP_ROWS // NS
QUAD = 4
GROUP = 16
NGROUP = ROWS_PER_SUB_E // GROUP

RB = 2048
RBK = RB * HH // 128
NBLK = NPK // RBK


def _segsum_body(hp_flat, srcb, dstb, out, idx_s, idx_d, rows, acc,
                 gsem0, gsem1, ssem0, ssem1, zsem):
  c = lax.axis_index("c")
  s = lax.axis_index("s")

  @pl.loop(0, EROW)
  def _(i):
    rows[0, 0, i, :] = jnp.zeros((16,), jnp.float32)

  zeros = [
      pltpu.async_copy(
          rows.at[0, 0], acc.at[pl.ds(s * ROWS_PER_SUB_N + k * EROW, EROW)],
          zsem)
      for k in range(ROWS_PER_SUB_N // EROW)
  ]
  for cp in zeros:
    cp.wait()

  plsc.subcore_barrier()

  base_row = s * ROWS_PER_SUB_E
  gsems = (gsem0, gsem1)
  ssems = (ssem0, ssem1)

  def load_group(g, slot):
    r0 = base_row + g * GROUP
    pltpu.sync_copy(srcb.at[pl.ds(r0, GROUP)], idx_s.at[slot])
    pltpu.sync_copy(dstb.at[pl.ds(r0, GROUP)], idx_d.at[slot])
    @pl.when(c == 1)
    def _():
      @pl.loop(0, GROUP)
      def _(j):
        for k in range(EROW // 16):
          sl = (slot, j, pl.ds(k * 16, 16))
          idx_s[sl] = idx_s[sl] + NP

  def fire_gathers(slot, q, bank):
    return [
        pltpu.async_copy(hp_flat.at[idx_s.at[slot, q * QUAD + j]],
                         rows.at[bank, j], gsems[bank])
        for j in range(QUAD)
    ]

  def fire_scatters(slot, q, bank):
    return [
        pltpu.async_copy(rows.at[bank, j],
                         acc.at[idx_d.at[slot, q * QUAD + j]],
                         ssems[bank], add=True)
        for j in range(QUAD)
    ]

  def drain(cps):
    for cp in cps:
      cp.wait()

  def wait_bank_scatters(bank):
    for j in range(QUAD):
      pltpu.make_async_copy(
          hp_flat.at[pl.ds(0, EROW)], rows.at[bank, j], ssems[bank]).wait()

  def process_group(g, slot, first):
    load_group(g, slot)
    ga = fire_gathers(slot, 0, 0)
    if not first:
      wait_bank_scatters(1)
    drain(ga)
    sa = fire_scatters(slot, 0, 0)
    gb = fire_gathers(slot, 1, 1)
    drain(gb)
    sb = fire_scatters(slot, 1, 1)
    drain(sa)
    gc = fire_gathers(slot, 2, 0)
    drain(gc)
    sc = fire_scatters(slot, 2, 0)
    drain(sb)
    gd = fire_gathers(slot, 3, 1)
    drain(gd)
    fire_scatters(slot, 3, 1)
    drain(sc)

  process_group(0, 0, True)

  @pl.loop(1, NGROUP)
  def _(g):
    process_group(g, lax.rem(g, 2), False)

  wait_bank_scatters(1)
  plsc.subcore_barrier()
  pltpu.sync_copy(
      acc.at[pl.ds(s * ROWS_PER_SUB_N, ROWS_PER_SUB_N)],
      out.at[c].at[pl.ds(s * ROWS_PER_SUB_N, ROWS_PER_SUB_N)],
  )


@functools.cache
def _build_segsum():
  return pl.kernel(
      _segsum_body,
      out_type=jax.ShapeDtypeStruct((NC, NP, HH), jnp.float32),
      mesh=plsc.VectorSubcoreMesh(
          core_axis_name="c", subcore_axis_name="s",
          num_cores=NC, num_subcores=NS
      ),
      scratch_types=[
          pltpu.VMEM((2, GROUP, EROW), jnp.int32),
          pltpu.VMEM((2, GROUP, EROW), jnp.int32),
          pltpu.VMEM((2, QUAD, EROW, HH), jnp.float32),
          pltpu.VMEM_SHARED((NP, HH), jnp.float32),
          pltpu.SemaphoreType.DMA,
          pltpu.SemaphoreType.DMA,
          pltpu.SemaphoreType.DMA,
          pltpu.SemaphoreType.DMA,
          pltpu.SemaphoreType.DMA,
      ],
      compiler_params=pltpu.CompilerParams(use_tc_tiling_on_sc=False),
  )


def _segsum(hp_flat, srcb, dstb):
  return _build_segsum()(hp_flat, srcb, dstb)


def _pack(y):
  def pack_half(h16):
    return jnp.concatenate([h16[k * RBK:(k + 1) * RBK] for k in range(8)],
                           axis=1)
  return pack_half(y[:, :HH]), pack_half(y[:, HH:])


def _unpack(p_ref):
  def unpack_half(p):
    return jnp.concatenate([p[:, k * HH:(k + 1) * HH] for k in range(8)],
                           axis=0)
  return jnp.concatenate([unpack_half(p_ref[0]), unpack_half(p_ref[1])],
                         axis=1)


def _first_body(x_ref, w_ref, b_ref, o_ref):
  y = jnp.dot(x_ref[...], w_ref[...], preferred_element_type=jnp.float32)
  y = jnp.maximum(y + b_ref[...], 0.0)
  lo, hi = _pack(y)
  o_ref[0] = lo
  o_ref[1] = hi


_first = pl.pallas_call(
    _first_body,
    out_shape=jax.ShapeDtypeStruct((NC, NPK, 128), jnp.float32),
    grid=(NBLK,),
    in_specs=[
        pl.BlockSpec((RB, F_IN), lambda i: (i, 0)),
        pl.BlockSpec((F_IN, H), lambda i: (0, 0)),
        pl.BlockSpec((1, H), lambda i: (0, 0)),
    ],
    out_specs=pl.BlockSpec((NC, RBK, 128), lambda i: (0, i, 0)),
)


def _conv_body(agg_ref, h_ref, hf_ref, wrel_ref, brel_ref, wroot_ref, fw_ref,
               o_ref, *, fuse_idx):
  agg = _unpack(agg_ref)
  h = _unpack(h_ref)
  hf = _unpack(hf_ref)
  y = jnp.dot(agg, wrel_ref[...], preferred_element_type=jnp.float32)
  y = y + brel_ref[...]
  y = y + jnp.dot(h, wroot_ref[...], preferred_element_type=jnp.float32)
  y = jnp.maximum(y, 0.0) + fw_ref[0, fuse_idx] * hf
  lo, hi = _pack(y)
  o_ref[0] = lo
  o_ref[1] = hi


def _make_conv(fuse_idx):
  return pl.pallas_call(
      functools.partial(_conv_body, fuse_idx=fuse_idx),
      out_shape=jax.ShapeDtypeStruct((NC, NPK, 128), jnp.float32),
      grid=(NBLK,),
      in_specs=[
          pl.BlockSpec((NC, RBK, 128), lambda i: (0, i, 0)),
          pl.BlockSpec((NC, RBK, 128), lambda i: (0, i, 0)),
          pl.BlockSpec((NC, RBK, 128), lambda i: (0, i, 0)),
          pl.BlockSpec((H, H), lambda i: (0, 0)),
          pl.BlockSpec((1, H), lambda i: (0, 0)),
          pl.BlockSpec((H, H), lambda i: (0, 0)),
          pl.BlockSpec((1, 2), lambda i: (0, 0)),
      ],
      out_specs=pl.BlockSpec((NC, RBK, 128), lambda i: (0, i, 0)),
  )


_conv1 = _make_conv(0)
_conv2 = _make_conv(1)


def _out_body(h2_ref, wout_ref, bout_ref, o_ref):
  h2 = _unpack(h2_ref)
  logits = jnp.dot(h2, wout_ref[...], preferred_element_type=jnp.float32)
  logits = logits + bout_ref[...]
  m = jnp.max(logits, axis=1, keepdims=True)
  lse = m + jnp.log(jnp.sum(jnp.exp(logits - m), axis=1, keepdims=True))
  o_ref[...] = logits - lse


_out = pl.pallas_call(
    _out_body,
    out_shape=jax.ShapeDtypeStruct((N, C), jnp.float32),
    grid=(NBLK,),
    in_specs=[
        pl.BlockSpec((NC, RBK, 128), lambda i: (0, i, 0)),
        pl.BlockSpec((H, C), lambda i: (0, 0)),
        pl.BlockSpec((1, C), lambda i: (0, 0)),
    ],
    out_specs=pl.BlockSpec((RB, C), lambda i: (i, 0)),
)


def kernel(x, edge_index, W_first, b_first, W_rel1, b_rel1, W_root1,
           W_rel2, b_rel2, W_root2, fuse_weight, W_out, b_out):
  EP = EP_ROWS * EROW

  def perm(n):
    return (n & ~jnp.int32(2047)) + (n & 255) * 8 + ((n >> 8) & 7)

  src = perm(edge_index[0])
  dst = perm(edge_index[1])
  src_r = jnp.concatenate([src, jnp.zeros((EP - E,), jnp.int32)]
                          ).reshape(EP_ROWS, EROW)
  dst_r = jnp.concatenate([dst, jnp.full((EP - E,), perm(jnp.int32(N))),]
                          ).reshape(EP_ROWS, EROW)

  b_first2 = b_first.reshape(1, H)
  b_rel1_2 = b_rel1.reshape(1, H)
  b_rel2_2 = b_rel2.reshape(1, H)
  fw2 = fuse_weight.reshape(1, 2)
  b_out2 = b_out.reshape(1, C)

  hp = _first(x, W_first.T, b_first2)
  agg1 = _segsum(hp.reshape(NC * NP, HH), src_r, dst_r).reshape(NC, NPK, 128)
  h1p = _conv1(agg1, hp, hp, W_rel1.T, b_rel1_2, W_root1.T, fw2)
  agg2 = _segsum(h1p.reshape(NC * NP, HH), src_r, dst_r).reshape(NC, NPK, 128)
  h2p = _conv2(agg2, h1p, hp, W_rel2.T, b_rel2_2, W_root2.T, fw2)
  return _out(h2p, W_out.T, b_out2)


# speedup vs baseline: 15.4108x; 1.3225x over previous
import functools

import jax
import jax.numpy as jnp
from jax import lax
from jax.experimental import pallas as pl
from jax.experimental.pallas import tpu as pltpu
from jax.experimental.pallas import tpu_sc as plsc

N = 100000
E = 1600000
F_IN = 128
H = 32
HH = 16
C = 2

NC = 2
NS = 16

ROWS_PER_SUB_N = 6272
NP = NS * ROWS_PER_SUB_N
NPK = NP * HH // 128
EROW = 128
EP_ROWS = 12544
ROWS_PER_SUB_E = EP_ROWS // NS
QUAD = 4
GROUP = 16
NGROUP = ROWS_PER_SUB_E // GROUP

RB = 2048
RBK = RB * HH // 128
NBLK = NPK // RBK


def _segsum_body(hp_flat, srcb, dstb, out, idx_s, idx_d, rows, acc,
                 gsem0, gsem1, ssem0, ssem1, isem):
  c = lax.axis_index("c")
  s = lax.axis_index("s")

  @pl.loop(0, EROW)
  def _(i):
    rows[0, 0, i, :] = jnp.zeros((16,), jnp.float32)

  zeros = [
      pltpu.async_copy(
          rows.at[0, 0], acc.at[pl.ds(s * ROWS_PER_SUB_N + k * EROW, EROW)],
          isem)
      for k in range(ROWS_PER_SUB_N // EROW)
  ]
  for cp in zeros:
    cp.wait()

  plsc.subcore_barrier()

  base_row = s * ROWS_PER_SUB_E
  gsems = (gsem0, gsem1)
  ssems = (ssem0, ssem1)

  def fire_idx(g, slot):
    r0 = base_row + g * GROUP
    return [
        pltpu.async_copy(srcb.at[pl.ds(r0, GROUP)], idx_s.at[slot], isem),
        pltpu.async_copy(dstb.at[pl.ds(r0, GROUP)], idx_d.at[slot], isem),
    ]

  def adjust(slot):
    @pl.when(c == 1)
    def _():
      @pl.loop(0, GROUP)
      def _(j):
        for k in range(EROW // 16):
          sl = (slot, j, pl.ds(k * 16, 16))
          idx_s[sl] = idx_s[sl] + NP

  def fire_gathers(slot, q, bank):
    return [
        pltpu.async_copy(hp_flat.at[idx_s.at[slot, q * QUAD + j]],
                         rows.at[bank, j], gsems[bank])
        for j in range(QUAD)
    ]

  def fire_scatters(slot, q, bank):
    return [
        pltpu.async_copy(rows.at[bank, j],
                         acc.at[idx_d.at[slot, q * QUAD + j]],
                         ssems[bank], add=True)
        for j in range(QUAD)
    ]

  def drain(cps):
    for cp in cps:
      cp.wait()

  def wait_bank_scatters(bank):
    for j in range(QUAD):
      pltpu.make_async_copy(
          hp_flat.at[pl.ds(0, EROW)], rows.at[bank, j], ssems[bank]).wait()

  def process_group(g, slot, first):
    ga = fire_gathers(slot, 0, 0)
    if not first:
      wait_bank_scatters(1)
    nxt = fire_idx(jnp.minimum(g + 1, NGROUP - 1), slot ^ 1)
    drain(ga)
    sa = fire_scatters(slot, 0, 0)
    gb = fire_gathers(slot, 1, 1)
    drain(gb)
    sb = fire_scatters(slot, 1, 1)
    drain(sa)
    gc = fire_gathers(slot, 2, 0)
    drain(gc)
    sc = fire_scatters(slot, 2, 0)
    drain(sb)
    gd = fire_gathers(slot, 3, 1)
    drain(gd)
    fire_scatters(slot, 3, 1)
    drain(sc)
    drain(nxt)
    adjust(slot ^ 1)

  drain(fire_idx(0, 0))
  adjust(0)
  process_group(0, 0, True)

  @pl.loop(1, NGROUP)
  def _(g):
    process_group(g, lax.rem(g, 2), False)

  wait_bank_scatters(1)
  plsc.subcore_barrier()
  pltpu.sync_copy(
      acc.at[pl.ds(s * ROWS_PER_SUB_N, ROWS_PER_SUB_N)],
      out.at[c].at[pl.ds(s * ROWS_PER_SUB_N, ROWS_PER_SUB_N)],
  )


@functools.cache
def _build_segsum():
  return pl.kernel(
      _segsum_body,
      out_type=jax.ShapeDtypeStruct((NC, NP, HH), jnp.float32),
      mesh=plsc.VectorSubcoreMesh(
          core_axis_name="c", subcore_axis_name="s",
          num_cores=NC, num_subcores=NS
      ),
      scratch_types=[
          pltpu.VMEM((2, GROUP, EROW), jnp.int32),
          pltpu.VMEM((2, GROUP, EROW), jnp.int32),
          pltpu.VMEM((2, QUAD, EROW, HH), jnp.float32),
          pltpu.VMEM_SHARED((NP, HH), jnp.float32),
          pltpu.SemaphoreType.DMA,
          pltpu.SemaphoreType.DMA,
          pltpu.SemaphoreType.DMA,
          pltpu.SemaphoreType.DMA,
          pltpu.SemaphoreType.DMA,
      ],
      compiler_params=pltpu.CompilerParams(use_tc_tiling_on_sc=False),
  )


def _segsum(hp_flat, srcb, dstb):
  return _build_segsum()(hp_flat, srcb, dstb)


def _pack(y):
  def pack_half(h16):
    return jnp.concatenate([h16[k * RBK:(k + 1) * RBK] for k in range(8)],
                           axis=1)
  return pack_half(y[:, :HH]), pack_half(y[:, HH:])


def _unpack(p_ref):
  def unpack_half(p):
    return jnp.concatenate([p[:, k * HH:(k + 1) * HH] for k in range(8)],
                           axis=0)
  return jnp.concatenate([unpack_half(p_ref[0]), unpack_half(p_ref[1])],
                         axis=1)


def _first_body(x_ref, w_ref, b_ref, o_ref):
  y = jnp.dot(x_ref[...], w_ref[...], preferred_element_type=jnp.float32)
  y = jnp.maximum(y + b_ref[...], 0.0)
  row = pl.program_id(0) * RB + lax.broadcasted_iota(jnp.int32, (RB, 1), 0)
  y = jnp.where(row < N, y, 0.0)
  lo, hi = _pack(y)
  o_ref[0] = lo
  o_ref[1] = hi


_first = pl.pallas_call(
    _first_body,
    out_shape=jax.ShapeDtypeStruct((NC, NPK, 128), jnp.float32),
    grid=(NBLK,),
    in_specs=[
        pl.BlockSpec((RB, F_IN), lambda i: (i, 0)),
        pl.BlockSpec((F_IN, H), lambda i: (0, 0)),
        pl.BlockSpec((1, H), lambda i: (0, 0)),
    ],
    out_specs=pl.BlockSpec((NC, RBK, 128), lambda i: (0, i, 0)),
)


def _conv_body(agg_ref, h_ref, hf_ref, wrelbd_ref, brelp_ref, wrootbd_ref,
               fw_ref, o_ref, *, fuse_idx):
  fw = fw_ref[0, fuse_idx]
  for j in range(NC):
    y = brelp_ref[j].reshape(1, 128)
    for i in range(NC):
      y = y + jnp.dot(agg_ref[i], wrelbd_ref[i, j],
                      preferred_element_type=jnp.float32)
      y = y + jnp.dot(h_ref[i], wrootbd_ref[i, j],
                      preferred_element_type=jnp.float32)
    o_ref[j] = jnp.maximum(y, 0.0) + fw * hf_ref[j]


def _make_conv(fuse_idx):
  return pl.pallas_call(
      functools.partial(_conv_body, fuse_idx=fuse_idx),
      out_shape=jax.ShapeDtypeStruct((NC, NPK, 128), jnp.float32),
      grid=(NBLK,),
      in_specs=[
          pl.BlockSpec((NC, RBK, 128), lambda i: (0, i, 0)),
          pl.BlockSpec((NC, RBK, 128), lambda i: (0, i, 0)),
          pl.BlockSpec((NC, RBK, 128), lambda i: (0, i, 0)),
          pl.BlockSpec((NC, NC, 128, 128), lambda i: (0, 0, 0, 0)),
          pl.BlockSpec((NC, 128), lambda i: (0, 0)),
          pl.BlockSpec((NC, NC, 128, 128), lambda i: (0, 0, 0, 0)),
          pl.BlockSpec((1, 2), lambda i: (0, 0)),
      ],
      out_specs=pl.BlockSpec((NC, RBK, 128), lambda i: (0, i, 0)),
  )


_conv1 = _make_conv(0)
_conv2 = _make_conv(1)


def _out_body(h2_ref, wout_ref, bout_ref, o_ref):
  h2 = _unpack(h2_ref)
  logits = jnp.dot(h2, wout_ref[...], preferred_element_type=jnp.float32)
  logits = logits + bout_ref[...]
  m = jnp.max(logits, axis=1, keepdims=True)
  lse = m + jnp.log(jnp.sum(jnp.exp(logits - m), axis=1, keepdims=True))
  o_ref[...] = logits - lse


_out = pl.pallas_call(
    _out_body,
    out_shape=jax.ShapeDtypeStruct((N, C), jnp.float32),
    grid=(NBLK,),
    in_specs=[
        pl.BlockSpec((NC, RBK, 128), lambda i: (0, i, 0)),
        pl.BlockSpec((H, C), lambda i: (0, 0)),
        pl.BlockSpec((1, C), lambda i: (0, 0)),
    ],
    out_specs=pl.BlockSpec((RB, C), lambda i: (i, 0)),
)


def kernel(x, edge_index, W_first, b_first, W_rel1, b_rel1, W_root1,
           W_rel2, b_rel2, W_root2, fuse_weight, W_out, b_out):
  EP = EP_ROWS * EROW

  def perm(n):
    return (n & ~jnp.int32(2047)) + (n & 255) * 8 + ((n >> 8) & 7)

  src = perm(edge_index[0])
  dst = perm(edge_index[1])
  src_r = jnp.concatenate([src, jnp.zeros((EP - E,), jnp.int32)]
                          ).reshape(EP_ROWS, EROW)
  dst_r = jnp.concatenate([dst, jnp.full((EP - E,), perm(jnp.int32(N))),]
                          ).reshape(EP_ROWS, EROW)

  def bd(wt):
    eye8 = jnp.eye(8, dtype=jnp.float32)
    return jnp.stack([
        jnp.stack([jnp.kron(eye8, wt[i * HH:(i + 1) * HH, j * HH:(j + 1) * HH])
                   for j in range(NC)])
        for i in range(NC)])

  def packb(b):
    return jnp.stack([jnp.tile(b[j * HH:(j + 1) * HH], 8) for j in range(NC)])

  b_first2 = b_first.reshape(1, H)
  fw2 = fuse_weight.reshape(1, 2)
  b_out2 = b_out.reshape(1, C)

  hp = _first(x, W_first.T, b_first2)
  agg1 = _segsum(hp.reshape(NC * NP, HH), src_r, dst_r).reshape(NC, NPK, 128)
  h1p = _conv1(agg1, hp, hp, bd(W_rel1.T), packb(b_rel1), bd(W_root1.T), fw2)
  agg2 = _segsum(h1p.reshape(NC * NP, HH), src_r, dst_r).reshape(NC, NPK, 128)
  h2p = _conv2(agg2, h1p, hp, bd(W_rel2.T), packb(b_rel2), bd(W_root2.T), fw2)
  return _out(h2p, W_out.T, b_out2)


# speedup vs baseline: 15.5396x; 1.0084x over previous
import functools

import jax
import jax.numpy as jnp
from jax import lax
from jax.experimental import pallas as pl
from jax.experimental.pallas import tpu as pltpu
from jax.experimental.pallas import tpu_sc as plsc

N = 100000
E = 1600000
F_IN = 128
H = 32
HH = 16
C = 2

NC = 2
NS = 16

ROWS_PER_SUB_N = 6272
NP = NS * ROWS_PER_SUB_N
NPK = NP * HH // 128
EROW = 128
EP_ROWS = 12544
ROWS_PER_SUB_E = EP_ROWS // NS
QUAD = 4
GROUP = 16
NGROUP = ROWS_PER_SUB_E // GROUP
SLAB = QUAD * EROW
SLABS_PER_GROUP = GROUP * EROW // SLAB
SLABS_PER_SUB = ROWS_PER_SUB_E * EROW // SLAB
EP_SLABS = EP_ROWS * EROW // SLAB

RB = 2048
RBK = RB * HH // 128
NBLK = NPK // RBK


def _segsum_body(hp_flat, srcb, dstb, out, idx_s, idx_d, rows, acc,
                 gsem0, gsem1, ssem0, ssem1, isem):
  c = lax.axis_index("c")
  s = lax.axis_index("s")

  @pl.loop(0, EROW)
  def _(i):
    rows[0, i, :] = jnp.zeros((16,), jnp.float32)

  zeros = [
      pltpu.async_copy(
          rows.at[0, pl.ds(0, EROW)],
          acc.at[pl.ds(s * ROWS_PER_SUB_N + k * EROW, EROW)],
          isem)
      for k in range(ROWS_PER_SUB_N // EROW)
  ]
  for cp in zeros:
    cp.wait()

  plsc.subcore_barrier()

  gsems = (gsem0, gsem1)
  ssems = (ssem0, ssem1)

  def fire_idx(g, slot):
    r0 = s * SLABS_PER_SUB + g * SLABS_PER_GROUP
    return [
        pltpu.async_copy(srcb.at[pl.ds(r0, SLABS_PER_GROUP)], idx_s.at[slot],
                         isem),
        pltpu.async_copy(dstb.at[pl.ds(r0, SLABS_PER_GROUP)], idx_d.at[slot],
                         isem),
    ]

  def adjust(slot):
    @pl.when(c == 1)
    def _():
      @pl.loop(0, SLABS_PER_GROUP)
      def _(j):
        for k in range(SLAB // 16):
          sl = (slot, j, pl.ds(k * 16, 16))
          idx_s[sl] = idx_s[sl] + NP

  def fire_gathers(slot, q, bank):
    return [
        pltpu.async_copy(hp_flat.at[idx_s.at[slot, q]], rows.at[bank],
                         gsems[bank])
    ]

  def fire_scatters(slot, q, bank):
    return [
        pltpu.async_copy(rows.at[bank], acc.at[idx_d.at[slot, q]],
                         ssems[bank], add=True)
    ]

  def drain(cps):
    for cp in cps:
      cp.wait()

  def wait_bank_scatters(bank):
    pltpu.make_async_copy(
        hp_flat.at[pl.ds(0, QUAD * EROW)], rows.at[bank], ssems[bank]).wait()

  def process_group(g, slot, first):
    ga = fire_gathers(slot, 0, 0)
    if not first:
      wait_bank_scatters(1)
    nxt = fire_idx(jnp.minimum(g + 1, NGROUP - 1), slot ^ 1)
    drain(ga)
    sa = fire_scatters(slot, 0, 0)
    gb = fire_gathers(slot, 1, 1)
    drain(gb)
    sb = fire_scatters(slot, 1, 1)
    drain(sa)
    gc = fire_gathers(slot, 2, 0)
    drain(gc)
    sc = fire_scatters(slot, 2, 0)
    drain(sb)
    gd = fire_gathers(slot, 3, 1)
    drain(gd)
    fire_scatters(slot, 3, 1)
    drain(sc)
    drain(nxt)
    adjust(slot ^ 1)

  drain(fire_idx(0, 0))
  adjust(0)
  process_group(0, 0, True)

  @pl.loop(1, NGROUP)
  def _(g):
    process_group(g, lax.rem(g, 2), False)

  wait_bank_scatters(1)
  plsc.subcore_barrier()
  pltpu.sync_copy(
      acc.at[pl.ds(s * ROWS_PER_SUB_N, ROWS_PER_SUB_N)],
      out.at[c].at[pl.ds(s * ROWS_PER_SUB_N, ROWS_PER_SUB_N)],
  )


@functools.cache
def _build_segsum():
  return pl.kernel(
      _segsum_body,
      out_type=jax.ShapeDtypeStruct((NC, NP, HH), jnp.float32),
      mesh=plsc.VectorSubcoreMesh(
          core_axis_name="c", subcore_axis_name="s",
          num_cores=NC, num_subcores=NS
      ),
      scratch_types=[
          pltpu.VMEM((2, SLABS_PER_GROUP, SLAB), jnp.int32),
          pltpu.VMEM((2, SLABS_PER_GROUP, SLAB), jnp.int32),
          pltpu.VMEM((2, SLAB, HH), jnp.float32),
          pltpu.VMEM_SHARED((NP, HH), jnp.float32),
          pltpu.SemaphoreType.DMA,
          pltpu.SemaphoreType.DMA,
          pltpu.SemaphoreType.DMA,
          pltpu.SemaphoreType.DMA,
          pltpu.SemaphoreType.DMA,
      ],
      compiler_params=pltpu.CompilerParams(use_tc_tiling_on_sc=False),
  )


def _segsum(hp_flat, srcb, dstb):
  return _build_segsum()(hp_flat, srcb, dstb)


def _pack(y):
  def pack_half(h16):
    return jnp.concatenate([h16[k * RBK:(k + 1) * RBK] for k in range(8)],
                           axis=1)
  return pack_half(y[:, :HH]), pack_half(y[:, HH:])


def _unpack(p_ref):
  def unpack_half(p):
    return jnp.concatenate([p[:, k * HH:(k + 1) * HH] for k in range(8)],
                           axis=0)
  return jnp.concatenate([unpack_half(p_ref[0]), unpack_half(p_ref[1])],
                         axis=1)


def _first_body(x_ref, w_ref, b_ref, o_ref):
  y = jnp.dot(x_ref[...], w_ref[...], preferred_element_type=jnp.float32)
  y = jnp.maximum(y + b_ref[...], 0.0)
  row = pl.program_id(0) * RB + lax.broadcasted_iota(jnp.int32, (RB, 1), 0)
  y = jnp.where(row < N, y, 0.0)
  lo, hi = _pack(y)
  o_ref[0] = lo
  o_ref[1] = hi


_first = pl.pallas_call(
    _first_body,
    out_shape=jax.ShapeDtypeStruct((NC, NPK, 128), jnp.float32),
    grid=(NBLK,),
    in_specs=[
        pl.BlockSpec((RB, F_IN), lambda i: (i, 0)),
        pl.BlockSpec((F_IN, H), lambda i: (0, 0)),
        pl.BlockSpec((1, H), lambda i: (0, 0)),
    ],
    out_specs=pl.BlockSpec((NC, RBK, 128), lambda i: (0, i, 0)),
)


def _conv_body(agg_ref, h_ref, hf_ref, wrelbd_ref, brelp_ref, wrootbd_ref,
               fw_ref, o_ref, *, fuse_idx):
  fw = fw_ref[0, fuse_idx]
  for j in range(NC):
    y = brelp_ref[j].reshape(1, 128)
    for i in range(NC):
      y = y + jnp.dot(agg_ref[i], wrelbd_ref[i, j],
                      preferred_element_type=jnp.float32)
      y = y + jnp.dot(h_ref[i], wrootbd_ref[i, j],
                      preferred_element_type=jnp.float32)
    o_ref[j] = jnp.maximum(y, 0.0) + fw * hf_ref[j]


def _make_conv(fuse_idx):
  return pl.pallas_call(
      functools.partial(_conv_body, fuse_idx=fuse_idx),
      out_shape=jax.ShapeDtypeStruct((NC, NPK, 128), jnp.float32),
      grid=(NBLK,),
      in_specs=[
          pl.BlockSpec((NC, RBK, 128), lambda i: (0, i, 0)),
          pl.BlockSpec((NC, RBK, 128), lambda i: (0, i, 0)),
          pl.BlockSpec((NC, RBK, 128), lambda i: (0, i, 0)),
          pl.BlockSpec((NC, NC, 128, 128), lambda i: (0, 0, 0, 0)),
          pl.BlockSpec((NC, 128), lambda i: (0, 0)),
          pl.BlockSpec((NC, NC, 128, 128), lambda i: (0, 0, 0, 0)),
          pl.BlockSpec((1, 2), lambda i: (0, 0)),
      ],
      out_specs=pl.BlockSpec((NC, RBK, 128), lambda i: (0, i, 0)),
  )


_conv1 = _make_conv(0)
_conv2 = _make_conv(1)


def _out_body(h2_ref, wout_ref, bout_ref, o_ref):
  h2 = _unpack(h2_ref)
  logits = jnp.dot(h2, wout_ref[...], preferred_element_type=jnp.float32)
  logits = logits + bout_ref[...]
  m = jnp.max(logits, axis=1, keepdims=True)
  lse = m + jnp.log(jnp.sum(jnp.exp(logits - m), axis=1, keepdims=True))
  o_ref[...] = logits - lse


_out = pl.pallas_call(
    _out_body,
    out_shape=jax.ShapeDtypeStruct((N, C), jnp.float32),
    grid=(NBLK,),
    in_specs=[
        pl.BlockSpec((NC, RBK, 128), lambda i: (0, i, 0)),
        pl.BlockSpec((H, C), lambda i: (0, 0)),
        pl.BlockSpec((1, C), lambda i: (0, 0)),
    ],
    out_specs=pl.BlockSpec((RB, C), lambda i: (i, 0)),
)


def kernel(x, edge_index, W_first, b_first, W_rel1, b_rel1, W_root1,
           W_rel2, b_rel2, W_root2, fuse_weight, W_out, b_out):
  EP = EP_ROWS * EROW

  def perm(n):
    return (n & ~jnp.int32(2047)) + (n & 255) * 8 + ((n >> 8) & 7)

  src = perm(edge_index[0])
  dst = perm(edge_index[1])
  src_r = jnp.concatenate([src, jnp.zeros((EP - E,), jnp.int32)]
                          ).reshape(EP_SLABS, SLAB)
  dst_r = jnp.concatenate([dst, jnp.full((EP - E,), perm(jnp.int32(N))),]
                          ).reshape(EP_SLABS, SLAB)

  def bd(wt):
    eye8 = jnp.eye(8, dtype=jnp.float32)
    return jnp.stack([
        jnp.stack([jnp.kron(eye8, wt[i * HH:(i + 1) * HH, j * HH:(j + 1) * HH])
                   for j in range(NC)])
        for i in range(NC)])

  def packb(b):
    return jnp.stack([jnp.tile(b[j * HH:(j + 1) * HH], 8) for j in range(NC)])

  b_first2 = b_first.reshape(1, H)
  fw2 = fuse_weight.reshape(1, 2)
  b_out2 = b_out.reshape(1, C)

  hp = _first(x, W_first.T, b_first2)
  agg1 = _segsum(hp.reshape(NC * NP, HH), src_r, dst_r).reshape(NC, NPK, 128)
  h1p = _conv1(agg1, hp, hp, bd(W_rel1.T), packb(b_rel1), bd(W_root1.T), fw2)
  agg2 = _segsum(h1p.reshape(NC * NP, HH), src_r, dst_r).reshape(NC, NPK, 128)
  h2p = _conv2(agg2, h1p, hp, bd(W_rel2.T), packb(b_rel2), bd(W_root2.T), fw2)
  return _out(h2p, W_out.T, b_out2)


# speedup vs baseline: 16.2379x; 1.0449x over previous
import functools

import jax
import jax.numpy as jnp
from jax import lax
from jax.experimental import pallas as pl
from jax.experimental.pallas import tpu as pltpu
from jax.experimental.pallas import tpu_sc as plsc

N = 100000
E = 1600000
F_IN = 128
H = 32
HH = 16
C = 2

NC = 2
NS = 16

ROWS_PER_SUB_N = 6272
NP = NS * ROWS_PER_SUB_N
NPK = NP * HH // 128
EROW = 128
EP_ROWS = 12544
ROWS_PER_SUB_E = EP_ROWS // NS
QUAD = 4
GROUP = 16
NGROUP = ROWS_PER_SUB_E // GROUP
SLAB = QUAD * EROW
SLABS_PER_GROUP = GROUP * EROW // SLAB
SLABS_PER_SUB = ROWS_PER_SUB_E * EROW // SLAB
EP_SLABS = EP_ROWS * EROW // SLAB

RB = 2048
RBK = RB * HH // 128
NBLK = NPK // RBK


def _segsum_body(hp_flat, srcb, dstb, out, idx_s, idx_d, rows, acc,
                 gsem0, gsem1, ssem0, ssem1, isem):
  c = lax.axis_index("c")
  s = lax.axis_index("s")

  @pl.loop(0, EROW)
  def _(i):
    rows[0, i, :] = jnp.zeros((16,), jnp.float32)

  zeros = [
      pltpu.async_copy(
          rows.at[0, pl.ds(0, EROW)],
          acc.at[pl.ds(s * ROWS_PER_SUB_N + k * EROW, EROW)],
          isem)
      for k in range(ROWS_PER_SUB_N // EROW)
  ]
  for cp in zeros:
    cp.wait()

  plsc.subcore_barrier()

  gsems = (gsem0, gsem1)
  ssems = (ssem0, ssem1)

  def fire_idx(g, slot):
    r0 = s * SLABS_PER_SUB + g * SLABS_PER_GROUP
    return [
        pltpu.async_copy(srcb.at[pl.ds(r0, SLABS_PER_GROUP)], idx_s.at[slot],
                         isem),
        pltpu.async_copy(dstb.at[pl.ds(r0, SLABS_PER_GROUP)], idx_d.at[slot],
                         isem),
    ]

  def adjust(slot):
    @pl.when(c == 1)
    def _():
      @pl.loop(0, SLABS_PER_GROUP)
      def _(j):
        for k in range(SLAB // 16):
          sl = (slot, j, pl.ds(k * 16, 16))
          idx_s[sl] = idx_s[sl] + NP

  def fire_gathers(slot, q, bank):
    return [
        pltpu.async_copy(hp_flat.at[idx_s.at[slot, q]], rows.at[bank],
                         gsems[bank])
    ]

  def fire_scatters(slot, q, bank):
    return [
        pltpu.async_copy(rows.at[bank], acc.at[idx_d.at[slot, q]],
                         ssems[bank], add=True)
    ]

  def drain(cps):
    for cp in cps:
      cp.wait()

  def wait_bank_scatters(bank):
    pltpu.make_async_copy(
        hp_flat.at[pl.ds(0, QUAD * EROW)], rows.at[bank], ssems[bank]).wait()

  def process_group(g, slot, first):
    ga = fire_gathers(slot, 0, 0)
    if not first:
      wait_bank_scatters(1)
    nxt = fire_idx(jnp.minimum(g + 1, NGROUP - 1), slot ^ 1)
    drain(ga)
    sa = fire_scatters(slot, 0, 0)
    gb = fire_gathers(slot, 1, 1)
    drain(gb)
    sb = fire_scatters(slot, 1, 1)
    drain(sa)
    gc = fire_gathers(slot, 2, 0)
    drain(gc)
    sc = fire_scatters(slot, 2, 0)
    drain(sb)
    gd = fire_gathers(slot, 3, 1)
    drain(gd)
    fire_scatters(slot, 3, 1)
    drain(sc)
    drain(nxt)
    adjust(slot ^ 1)

  drain(fire_idx(0, 0))
  adjust(0)
  process_group(0, 0, True)

  @pl.loop(1, NGROUP)
  def _(g):
    process_group(g, lax.rem(g, 2), False)

  wait_bank_scatters(1)
  plsc.subcore_barrier()
  pltpu.sync_copy(
      acc.at[pl.ds(s * ROWS_PER_SUB_N, ROWS_PER_SUB_N)],
      out.at[c].at[pl.ds(s * ROWS_PER_SUB_N, ROWS_PER_SUB_N)],
  )


@functools.cache
def _build_segsum():
  return pl.kernel(
      _segsum_body,
      out_type=jax.ShapeDtypeStruct((NC, NP, HH), jnp.float32),
      mesh=plsc.VectorSubcoreMesh(
          core_axis_name="c", subcore_axis_name="s",
          num_cores=NC, num_subcores=NS
      ),
      scratch_types=[
          pltpu.VMEM((2, SLABS_PER_GROUP, SLAB), jnp.int32),
          pltpu.VMEM((2, SLABS_PER_GROUP, SLAB), jnp.int32),
          pltpu.VMEM((2, SLAB, HH), jnp.float32),
          pltpu.VMEM_SHARED((NP, HH), jnp.float32),
          pltpu.SemaphoreType.DMA,
          pltpu.SemaphoreType.DMA,
          pltpu.SemaphoreType.DMA,
          pltpu.SemaphoreType.DMA,
          pltpu.SemaphoreType.DMA,
      ],
      compiler_params=pltpu.CompilerParams(use_tc_tiling_on_sc=False),
  )


def _segsum(hp_flat, srcb, dstb):
  return _build_segsum()(hp_flat, srcb, dstb)


def _pack(y):
  def pack_half(h16):
    return jnp.concatenate([h16[k * RBK:(k + 1) * RBK] for k in range(8)],
                           axis=1)
  return pack_half(y[:, :HH]), pack_half(y[:, HH:])


def _unpack(p_ref):
  def unpack_half(p):
    return jnp.concatenate([p[:, k * HH:(k + 1) * HH] for k in range(8)],
                           axis=0)
  return jnp.concatenate([unpack_half(p_ref[0]), unpack_half(p_ref[1])],
                         axis=1)


def _first_body(x_ref, w_ref, b_ref, o_ref):
  y = jnp.dot(x_ref[...], w_ref[...], preferred_element_type=jnp.float32)
  y = jnp.maximum(y + b_ref[...], 0.0)
  row = pl.program_id(0) * RB + lax.broadcasted_iota(jnp.int32, (RB, 1), 0)
  y = jnp.where(row < N, y, 0.0)
  lo, hi = _pack(y)
  o_ref[0] = lo
  o_ref[1] = hi


_first = pl.pallas_call(
    _first_body,
    out_shape=jax.ShapeDtypeStruct((NC, NPK, 128), jnp.float32),
    grid=(NBLK,),
    in_specs=[
        pl.BlockSpec((RB, F_IN), lambda i: (i, 0)),
        pl.BlockSpec((F_IN, H), lambda i: (0, 0)),
        pl.BlockSpec((1, H), lambda i: (0, 0)),
    ],
    out_specs=pl.BlockSpec((NC, RBK, 128), lambda i: (0, i, 0)),
)


def _conv_body(agg_ref, h_ref, hf_ref, wrelbd_ref, brelp_ref, wrootbd_ref,
               fw_ref, o_ref, *, fuse_idx):
  fw = fw_ref[0, fuse_idx]
  for j in range(NC):
    y = brelp_ref[j].reshape(1, 128)
    for i in range(NC):
      y = y + jnp.dot(agg_ref[i], wrelbd_ref[i, j],
                      preferred_element_type=jnp.float32)
      y = y + jnp.dot(h_ref[i], wrootbd_ref[i, j],
                      preferred_element_type=jnp.float32)
    o_ref[j] = jnp.maximum(y, 0.0) + fw * hf_ref[j]


def _make_conv(fuse_idx):
  return pl.pallas_call(
      functools.partial(_conv_body, fuse_idx=fuse_idx),
      out_shape=jax.ShapeDtypeStruct((NC, NPK, 128), jnp.float32),
      grid=(NBLK,),
      in_specs=[
          pl.BlockSpec((NC, RBK, 128), lambda i: (0, i, 0)),
          pl.BlockSpec((NC, RBK, 128), lambda i: (0, i, 0)),
          pl.BlockSpec((NC, RBK, 128), lambda i: (0, i, 0)),
          pl.BlockSpec((NC, NC, 128, 128), lambda i: (0, 0, 0, 0)),
          pl.BlockSpec((NC, 128), lambda i: (0, 0)),
          pl.BlockSpec((NC, NC, 128, 128), lambda i: (0, 0, 0, 0)),
          pl.BlockSpec((1, 2), lambda i: (0, 0)),
      ],
      out_specs=pl.BlockSpec((NC, RBK, 128), lambda i: (0, i, 0)),
  )


_conv1 = _make_conv(0)
_conv2 = _make_conv(1)


def _out_body(h2_ref, woutbd_ref, boutp_ref, o_ref):
  lp = boutp_ref[...]
  for i in range(NC):
    lp = lp + jnp.dot(h2_ref[i], woutbd_ref[i],
                      preferred_element_type=jnp.float32)
  logits = jnp.concatenate([lp[:, 2 * k:2 * k + 2] for k in range(8)], axis=0)
  m = jnp.max(logits, axis=1, keepdims=True)
  lse = m + jnp.log(jnp.sum(jnp.exp(logits - m), axis=1, keepdims=True))
  o_ref[...] = logits - lse


_out = pl.pallas_call(
    _out_body,
    out_shape=jax.ShapeDtypeStruct((N, C), jnp.float32),
    grid=(NBLK,),
    in_specs=[
        pl.BlockSpec((NC, RBK, 128), lambda i: (0, i, 0)),
        pl.BlockSpec((NC, 128, 2 * 8), lambda i: (0, 0, 0)),
        pl.BlockSpec((1, 2 * 8), lambda i: (0, 0)),
    ],
    out_specs=pl.BlockSpec((RB, C), lambda i: (i, 0)),
)


_PROW = 128
_PGRID = (EP_SLABS + _PROW - 1) // _PROW
_EROWS_REAL = E // SLAB
_DUMP = 98304 + (100000 & 255) * 8 + ((100000 >> 8) & 7)


def _perm_body(ei_ref, src_ref, dst_ref):
  rid = pl.program_id(0) * _PROW + lax.broadcasted_iota(jnp.int32, (_PROW, 1), 0)
  valid = rid < _EROWS_REAL

  def perm(n):
    return (n & ~jnp.int32(2047)) + (n & 255) * 8 + ((n >> 8) & 7)

  src_ref[...] = jnp.where(valid, perm(ei_ref[0]), 0)
  dst_ref[...] = jnp.where(valid, perm(ei_ref[1]), _DUMP)


_permpad = pl.pallas_call(
    _perm_body,
    out_shape=(jax.ShapeDtypeStruct((EP_SLABS, SLAB), jnp.int32),
               jax.ShapeDtypeStruct((EP_SLABS, SLAB), jnp.int32)),
    grid=(_PGRID,),
    in_specs=[pl.BlockSpec((2, _PROW, SLAB), lambda i: (0, i, 0))],
    out_specs=(pl.BlockSpec((_PROW, SLAB), lambda i: (i, 0)),
               pl.BlockSpec((_PROW, SLAB), lambda i: (i, 0))),
)


def kernel(x, edge_index, W_first, b_first, W_rel1, b_rel1, W_root1,
           W_rel2, b_rel2, W_root2, fuse_weight, W_out, b_out):
  src_r, dst_r = _permpad(edge_index.reshape(2, _EROWS_REAL, SLAB))

  def bd(wt):
    eye8 = jnp.eye(8, dtype=jnp.float32)
    return jnp.stack([
        jnp.stack([jnp.kron(eye8, wt[i * HH:(i + 1) * HH, j * HH:(j + 1) * HH])
                   for j in range(NC)])
        for i in range(NC)])

  def packb(b):
    return jnp.stack([jnp.tile(b[j * HH:(j + 1) * HH], 8) for j in range(NC)])

  b_first2 = b_first.reshape(1, H)
  fw2 = fuse_weight.reshape(1, 2)
  eye8 = jnp.eye(8, dtype=jnp.float32)
  woutbd = jnp.stack(
      [jnp.kron(eye8, W_out[:, i * HH:(i + 1) * HH].T) for i in range(NC)])
  boutp = jnp.tile(b_out, 8).reshape(1, 16)

  hp = _first(x, W_first.T, b_first2)
  agg1 = _segsum(hp.reshape(NC * NP, HH), src_r, dst_r).reshape(NC, NPK, 128)
  h1p = _conv1(agg1, hp, hp, bd(W_rel1.T), packb(b_rel1), bd(W_root1.T), fw2)
  agg2 = _segsum(h1p.reshape(NC * NP, HH), src_r, dst_r).reshape(NC, NPK, 128)
  h2p = _conv2(agg2, h1p, hp, bd(W_rel2.T), packb(b_rel2), bd(W_root2.T), fw2)
  return _out(h2p, woutbd, boutp)
